# baseline (speedup 1.0000x reference)
import jax
import jax.numpy as jnp
from jax.experimental import pallas as pl
from jax.experimental.pallas import tpu as pltpu

_BM = 400


def _xw_kernel(x_ref, w_ref, h_ref):
    h_ref[...] = jnp.dot(x_ref[...], w_ref[...],
                         preferred_element_type=jnp.float32)


def _spmm_kernel(adj_ref, h_ref, out_ref):
    out_ref[...] = jnp.maximum(
        jnp.dot(adj_ref[...], h_ref[...],
                preferred_element_type=jnp.float32),
        0.0,
    )


def kernel(X, adj, W):
    n, in_dim = X.shape
    out_dim = W.shape[1]

    h = pl.pallas_call(
        _xw_kernel,
        out_shape=jax.ShapeDtypeStruct((n, out_dim), jnp.float32),
    )(X, W)

    return pl.pallas_call(
        _spmm_kernel,
        grid=(n // _BM,),
        in_specs=[
            pl.BlockSpec((_BM, n), lambda i: (i, 0)),
            pl.BlockSpec((n, out_dim), lambda i: (0, 0)),
        ],
        out_specs=pl.BlockSpec((_BM, out_dim), lambda i: (i, 0)),
        out_shape=jax.ShapeDtypeStruct((n, out_dim), jnp.float32),
        compiler_params=pltpu.CompilerParams(
            dimension_semantics=("parallel",),
        ),
    )(adj, h)


# speedup vs baseline: 1.0435x; 1.0435x over previous
import jax
import jax.numpy as jnp
from jax.experimental import pallas as pl
from jax.experimental.pallas import tpu as pltpu

_BM = 400


def _gcn_kernel(x_ref, w_ref, adj_ref, out_ref, h_ref):
    @pl.when(pl.program_id(0) == 0)
    def _():
        h = jnp.dot(x_ref[...], w_ref[...],
                    preferred_element_type=jnp.float32)
        h_ref[...] = h.astype(jnp.bfloat16)

    out_ref[...] = jnp.maximum(
        jnp.dot(adj_ref[...].astype(jnp.bfloat16), h_ref[...],
                preferred_element_type=jnp.float32),
        0.0,
    )


def kernel(X, adj, W):
    n, in_dim = X.shape
    out_dim = W.shape[1]

    return pl.pallas_call(
        _gcn_kernel,
        grid=(n // _BM,),
        in_specs=[
            pl.BlockSpec((n, in_dim), lambda i: (0, 0)),
            pl.BlockSpec((in_dim, out_dim), lambda i: (0, 0)),
            pl.BlockSpec((_BM, n), lambda i: (i, 0)),
        ],
        out_specs=pl.BlockSpec((_BM, out_dim), lambda i: (i, 0)),
        out_shape=jax.ShapeDtypeStruct((n, out_dim), jnp.float32),
        scratch_shapes=[pltpu.VMEM((n, out_dim), jnp.bfloat16)],
        compiler_params=pltpu.CompilerParams(
            dimension_semantics=("arbitrary",),
        ),
    )(X, W, adj)
